# baseline (speedup 1.0000x reference)
import functools

import jax
import jax.numpy as jnp
from jax import lax
from jax.experimental import pallas as pl
from jax.experimental.pallas import tpu as pltpu
from jax.experimental.pallas import tpu_sc as plsc

N_S = 8192
N_T = 32768
C = 22
TEMP = 2.0
KL_W = 0.2

BS = 256
BT = 32768
NSB = N_S // BS
NTB = N_T // BT
CP = 128


def _knn_body(q_ref, t_ref, out_ref, max_ref, idx_ref):
    j = pl.program_id(1)
    q = q_ref[...]
    t = t_ref[...]
    q_aug = jnp.where(
        lax.broadcasted_iota(jnp.int32, q.shape, 1) == 3, 1.0, q)
    tsqh = 0.5 * jnp.sum(t * t, axis=0, keepdims=True)
    t_aug = jnp.where(
        lax.broadcasted_iota(jnp.int32, t.shape, 0) == 3, -tsqh, t)
    RB = 64
    NRB = BS // RB
    NB = 8
    CB = BT // NB
    m = [None] * NRB
    mi = [None] * NRB
    for b in range(NB):
        sb = jnp.dot(q_aug, t_aug[:, b * CB:(b + 1) * CB],
                     preferred_element_type=jnp.float32)
        for rb in range(NRB):
            sub = sb[rb * RB:(rb + 1) * RB, :]
            for c in range(CB // 128):
                s = sub[:, c * 128:(c + 1) * 128]
                gc = b * (CB // 128) + c
                if gc == 0:
                    m[rb] = s
                    mi[rb] = jnp.zeros((RB, 128), jnp.int32)
                else:
                    upd = s > m[rb]
                    m[rb] = jnp.where(upd, s, m[rb])
                    mi[rb] = jnp.where(upd, gc, mi[rb])
    maxs, cands = [], []
    for rb in range(NRB):
        rmax = jnp.max(m[rb], axis=1, keepdims=True)
        lane = lax.broadcasted_iota(jnp.int32, (RB, 128), 1)
        gidx = mi[rb] * 128 + lane
        c_ = jnp.min(jnp.where(m[rb] == rmax, gidx, jnp.int32(2**30)),
                     axis=1, keepdims=True)
        maxs.append(rmax)
        cands.append(c_)
    rowmax = jnp.concatenate(maxs, axis=0)
    cand = jnp.concatenate(cands, axis=0) + j * BT

    @pl.when(j == 0)
    def _():
        max_ref[...] = rowmax
        idx_ref[...] = cand

    @pl.when(j > 0)
    def _():
        take = rowmax > max_ref[...]
        idx_ref[...] = jnp.where(take, cand, idx_ref[...])
        max_ref[...] = jnp.where(take, rowmax, max_ref[...])

    @pl.when(j == NTB - 1)
    def _():
        out_ref[0] = idx_ref[...]


def _knn_idx(q8, t8):
    idx3 = pl.pallas_call(
        _knn_body,
        grid=(NSB, NTB),
        in_specs=[
            pl.BlockSpec((BS, 8), lambda i, j: (i, 0)),
            pl.BlockSpec((8, BT), lambda i, j: (0, j)),
        ],
        out_specs=pl.BlockSpec((1, BS, 1), lambda i, j: (i, 0, 0)),
        out_shape=jax.ShapeDtypeStruct((NSB, BS, 1), jnp.int32),
        scratch_shapes=[
            pltpu.VMEM((BS, 1), jnp.float32),
            pltpu.VMEM((BS, 1), jnp.int32),
        ],
    )(q8, t8)
    return idx3.reshape(N_S)


def _make_sc_gather():
    info = plsc.get_sparse_core_info()
    nw = info.num_cores * info.num_subcores
    b_per_w = N_S // nw
    n_chunks = max(1, b_per_w // 128)
    chunk = b_per_w // n_chunks
    mesh = plsc.VectorSubcoreMesh(core_axis_name="c", subcore_axis_name="s")

    @functools.partial(
        pl.kernel,
        mesh=mesh,
        out_type=jax.ShapeDtypeStruct((N_S, CP), jnp.float32),
        scratch_types=[
            pltpu.VMEM((b_per_w,), jnp.int32),
            pltpu.VMEM((b_per_w, CP), jnp.float32),
            pltpu.SemaphoreType.DMA,
        ],
    )
    def gather_k(table_hbm, idx_hbm, out_hbm, idx_v, rows_v, sem):
        wid = lax.axis_index("s") * info.num_cores + lax.axis_index("c")
        base = wid * b_per_w
        pltpu.sync_copy(idx_hbm.at[pl.ds(base, b_per_w)], idx_v)
        copies = []
        for ci in range(n_chunks):
            copies.append(pltpu.async_copy(
                table_hbm.at[idx_v.at[pl.ds(ci * chunk, chunk)]],
                rows_v.at[pl.ds(ci * chunk, chunk)],
                sem,
            ))
        for cp in copies:
            cp.wait()
        pltpu.sync_copy(rows_v, out_hbm.at[pl.ds(base, b_per_w)])

    return gather_k


def _kl_body(m_ref, s_ref, out_ref):
    m = m_ref[...]
    s = s_ref[...]
    mask = lax.broadcasted_iota(jnp.int32, m.shape, 1) < C
    neg = jnp.float32(-jnp.inf)

    zm = jnp.where(mask, m * (1.0 / TEMP), neg)
    mm = jnp.max(zm, axis=1, keepdims=True)
    em = jnp.exp(zm - mm)
    p = em / jnp.sum(em, axis=1, keepdims=True)

    zs = jnp.where(mask, s * (1.0 / TEMP), neg)
    ms = jnp.max(zs, axis=1, keepdims=True)
    es = jnp.exp(zs - ms)
    logp = (zs - ms) - jnp.log(jnp.sum(es, axis=1, keepdims=True))

    plogp = jnp.where(p > 0, p * jnp.log(jnp.where(p > 0, p, 1.0)), 0.0)
    term = jnp.where(mask, plogp - p * logp, 0.0)
    total = jnp.sum(jnp.sum(term, axis=1, keepdims=True), axis=0, keepdims=True)
    out_ref[...] = total * jnp.float32(KL_W * TEMP * TEMP / N_S)


def _kl_loss(matched, seg_pad):
    out = pl.pallas_call(
        _kl_body,
        out_shape=jax.ShapeDtypeStruct((1, 1), jnp.float32),
    )(matched, seg_pad)
    return out[0, 0]


def kernel(student_coords, teacher_coords, teacher_logits, seg_logits):
    q8 = jnp.pad(student_coords, ((0, 0), (0, 5)))
    t8 = jnp.pad(teacher_coords, ((0, 0), (0, 5))).T
    idx = _knn_idx(q8, t8)

    table = jnp.pad(teacher_logits, ((0, 0), (0, CP - C)))
    matched = _make_sc_gather()(table, idx)

    seg_pad = jnp.pad(seg_logits, ((0, 0), (0, CP - C)))
    return _kl_loss(matched, seg_pad)


# speedup vs baseline: 1.0027x; 1.0027x over previous
import functools

import jax
import jax.numpy as jnp
from jax import lax
from jax.experimental import pallas as pl
from jax.experimental.pallas import tpu as pltpu
from jax.experimental.pallas import tpu_sc as plsc

N_S = 8192
N_T = 32768
C = 22
TEMP = 2.0
KL_W = 0.2

BS = 256
BT = 32768
NSB = N_S // BS
NTB = N_T // BT
CP = 128


def _knn_body(q_ref, t_ref, out_ref):
    q = q_ref[...]
    t = t_ref[...]
    q_aug = jnp.where(
        lax.broadcasted_iota(jnp.int32, q.shape, 1) == 3, 1.0, q)
    tsqh = 0.5 * jnp.sum(t * t, axis=0, keepdims=True)
    t_aug = jnp.where(
        lax.broadcasted_iota(jnp.int32, t.shape, 0) == 3, -tsqh, t)
    RB = 64
    NRB = BS // RB
    NB = 8
    CB = BT // NB
    m = [None] * NRB
    mi = [None] * NRB
    for b in range(NB):
        sb = jnp.dot(q_aug, t_aug[:, b * CB:(b + 1) * CB],
                     preferred_element_type=jnp.float32)
        for rb in range(NRB):
            sub = sb[rb * RB:(rb + 1) * RB, :]
            for c in range(CB // 128):
                s = sub[:, c * 128:(c + 1) * 128]
                gc = b * (CB // 128) + c
                if gc == 0:
                    m[rb] = s
                    mi[rb] = jnp.zeros((RB, 128), jnp.int32)
                else:
                    upd = s > m[rb]
                    m[rb] = jnp.where(upd, s, m[rb])
                    mi[rb] = jnp.where(upd, gc, mi[rb])
    maxs, cands = [], []
    for rb in range(NRB):
        rmax = jnp.max(m[rb], axis=1, keepdims=True)
        lane = lax.broadcasted_iota(jnp.int32, (RB, 128), 1)
        gidx = mi[rb] * 128 + lane
        c_ = jnp.min(jnp.where(m[rb] == rmax, gidx, jnp.int32(2**30)),
                     axis=1, keepdims=True)
        maxs.append(rmax)
        cands.append(c_)
    out_ref[0] = jnp.concatenate(cands, axis=0)


def _knn_idx(q8, t8):
    idx3 = pl.pallas_call(
        _knn_body,
        grid=(NSB,),
        in_specs=[
            pl.BlockSpec((BS, 8), lambda i: (i, 0)),
            pl.BlockSpec((8, BT), lambda i: (0, 0)),
        ],
        out_specs=pl.BlockSpec((1, BS, 1), lambda i: (i, 0, 0)),
        out_shape=jax.ShapeDtypeStruct((NSB, BS, 1), jnp.int32),
    )(q8, t8)
    return idx3.reshape(N_S)


def _make_sc_gather():
    info = plsc.get_sparse_core_info()
    nw = info.num_cores * info.num_subcores
    b_per_w = N_S // nw
    n_chunks = max(1, b_per_w // 128)
    chunk = b_per_w // n_chunks
    mesh = plsc.VectorSubcoreMesh(core_axis_name="c", subcore_axis_name="s")

    @functools.partial(
        pl.kernel,
        mesh=mesh,
        out_type=jax.ShapeDtypeStruct((N_S, CP), jnp.float32),
        scratch_types=[
            pltpu.VMEM((b_per_w,), jnp.int32),
            pltpu.VMEM((b_per_w, CP), jnp.float32),
            pltpu.SemaphoreType.DMA,
        ],
    )
    def gather_k(table_hbm, idx_hbm, out_hbm, idx_v, rows_v, sem):
        wid = lax.axis_index("s") * info.num_cores + lax.axis_index("c")
        base = wid * b_per_w
        pltpu.sync_copy(idx_hbm.at[pl.ds(base, b_per_w)], idx_v)
        copies = []
        for ci in range(n_chunks):
            copies.append(pltpu.async_copy(
                table_hbm.at[idx_v.at[pl.ds(ci * chunk, chunk)]],
                rows_v.at[pl.ds(ci * chunk, chunk)],
                sem,
            ))
        for cp in copies:
            cp.wait()
        pltpu.sync_copy(rows_v, out_hbm.at[pl.ds(base, b_per_w)])

    return gather_k


def _kl_body(m_ref, s_ref, out_ref):
    m = m_ref[...]
    s = s_ref[...]
    bias = jnp.where(lax.broadcasted_iota(jnp.int32, (1, CP), 1) < C,
                     0.0, -1e30)

    zm = m * (1.0 / TEMP) + bias
    mm = jnp.max(zm, axis=1, keepdims=True)
    em = jnp.exp(zm - mm)
    sm = jnp.sum(em, axis=1, keepdims=True)
    p = em * (1.0 / sm)

    zs = s * (1.0 / TEMP) + bias
    ms = jnp.max(zs, axis=1, keepdims=True)
    es = jnp.exp(zs - ms)
    ss = jnp.sum(es, axis=1, keepdims=True)

    row = (jnp.sum(p * (zm - zs), axis=1, keepdims=True)
           + (ms + jnp.log(ss)) - (mm + jnp.log(sm)))
    total = jnp.sum(row, axis=0, keepdims=True)
    out_ref[...] = total * jnp.float32(KL_W * TEMP * TEMP / N_S)


def _kl_loss(matched, seg_pad):
    out = pl.pallas_call(
        _kl_body,
        out_shape=jax.ShapeDtypeStruct((1, 1), jnp.float32),
    )(matched, seg_pad)
    return out[0, 0]


def kernel(student_coords, teacher_coords, teacher_logits, seg_logits):
    q8 = jnp.pad(student_coords, ((0, 0), (0, 5)))
    t8 = jnp.pad(teacher_coords, ((0, 0), (0, 5))).T
    idx = _knn_idx(q8, t8)

    table = jnp.pad(teacher_logits, ((0, 0), (0, CP - C)))
    matched = _make_sc_gather()(table, idx)

    seg_pad = jnp.pad(seg_logits, ((0, 0), (0, CP - C)))
    return _kl_loss(matched, seg_pad)


# speedup vs baseline: 1.0193x; 1.0166x over previous
import functools

import jax
import jax.numpy as jnp
from jax import lax
from jax.experimental import pallas as pl
from jax.experimental.pallas import tpu as pltpu
from jax.experimental.pallas import tpu_sc as plsc

N_S = 8192
N_T = 32768
C = 22
TEMP = 2.0
KL_W = 0.2

BS = 256
BT = 32768
NSB = N_S // BS
NTB = N_T // BT
CP = 128


def _knn_body(q_ref, t_ref, out_ref):
    q = q_ref[...]
    t = t_ref[...]
    q_aug = jnp.where(
        lax.broadcasted_iota(jnp.int32, q.shape, 1) == 3, 1.0, q)
    tsqh = 0.5 * jnp.sum(t * t, axis=0, keepdims=True)
    t_aug = jnp.where(
        lax.broadcasted_iota(jnp.int32, t.shape, 0) == 3, -tsqh, t)
    RB = 64
    NRB = BS // RB
    NB = 8
    CB = BT // NB
    m = [None] * NRB
    mi = [None] * NRB
    for b in range(NB):
        sb = jnp.dot(q_aug, t_aug[:, b * CB:(b + 1) * CB],
                     preferred_element_type=jnp.float32)
        for rb in range(NRB):
            sub = sb[rb * RB:(rb + 1) * RB, :]
            for c in range(CB // 128):
                s = sub[:, c * 128:(c + 1) * 128]
                gc = b * (CB // 128) + c
                if gc == 0:
                    m[rb] = s
                    mi[rb] = jnp.zeros((RB, 128), jnp.int32)
                else:
                    upd = s > m[rb]
                    m[rb] = jnp.where(upd, s, m[rb])
                    mi[rb] = jnp.where(upd, gc, mi[rb])
    maxs, cands = [], []
    for rb in range(NRB):
        rmax = jnp.max(m[rb], axis=1, keepdims=True)
        lane = lax.broadcasted_iota(jnp.int32, (RB, 128), 1)
        gidx = mi[rb] * 128 + lane
        c_ = jnp.min(jnp.where(m[rb] == rmax, gidx, jnp.int32(2**30)),
                     axis=1, keepdims=True)
        maxs.append(rmax)
        cands.append(c_)
    out_ref[0] = jnp.concatenate(cands, axis=0)


def _knn_idx(q8, t8):
    idx3 = pl.pallas_call(
        _knn_body,
        grid=(NSB,),
        in_specs=[
            pl.BlockSpec((BS, 8), lambda i: (i, 0)),
            pl.BlockSpec((8, BT), lambda i: (0, 0)),
        ],
        out_specs=pl.BlockSpec((1, BS, 1), lambda i: (i, 0, 0)),
        out_shape=jax.ShapeDtypeStruct((NSB, BS, 1), jnp.int32),
    )(q8, t8)
    return idx3.reshape(N_S)


def _make_sc_gather():
    info = plsc.get_sparse_core_info()
    nw = info.num_cores * info.num_subcores
    b_per_w = N_S // nw
    n_chunks = max(1, b_per_w // 128)
    chunk = b_per_w // n_chunks
    mesh = plsc.VectorSubcoreMesh(core_axis_name="c", subcore_axis_name="s")

    @functools.partial(
        pl.kernel,
        mesh=mesh,
        out_type=jax.ShapeDtypeStruct((N_S, CP), jnp.float32),
        scratch_types=[
            pltpu.VMEM((b_per_w,), jnp.int32),
            pltpu.VMEM((b_per_w, CP), jnp.float32),
            pltpu.SemaphoreType.DMA,
        ],
    )
    def gather_k(table_hbm, idx_hbm, out_hbm, idx_v, rows_v, sem):
        wid = lax.axis_index("s") * info.num_cores + lax.axis_index("c")
        base = wid * b_per_w
        pltpu.sync_copy(idx_hbm.at[pl.ds(base, b_per_w)], idx_v)
        copies = []
        for ci in range(n_chunks):
            copies.append(pltpu.async_copy(
                table_hbm.at[idx_v.at[pl.ds(ci * chunk, chunk)]],
                rows_v.at[pl.ds(ci * chunk, chunk)],
                sem,
            ))
        for cp in copies:
            cp.wait()
        pltpu.sync_copy(rows_v, out_hbm.at[pl.ds(base, b_per_w)])

    return gather_k


def _kl_body(m_ref, s_ref, out_ref):
    m = jnp.transpose(m_ref[:, 0:24])
    s = jnp.transpose(s_ref[:, 0:24])
    bias = jnp.where(lax.broadcasted_iota(jnp.int32, (24, 1), 0) < C,
                     0.0, -1e30)

    zm = m * (1.0 / TEMP) + bias
    mm = jnp.max(zm, axis=0, keepdims=True)
    em = jnp.exp(zm - mm)
    sm = jnp.sum(em, axis=0, keepdims=True)
    p = em * (1.0 / sm)

    zs = s * (1.0 / TEMP) + bias
    ms = jnp.max(zs, axis=0, keepdims=True)
    es = jnp.exp(zs - ms)
    ss = jnp.sum(es, axis=0, keepdims=True)

    row = (jnp.sum(p * (zm - zs), axis=0, keepdims=True)
           + (ms + jnp.log(ss)) - (mm + jnp.log(sm)))
    total = jnp.sum(row, axis=1, keepdims=True)
    out_ref[...] = total * jnp.float32(KL_W * TEMP * TEMP / N_S)


def _kl_loss(matched, seg_pad):
    out = pl.pallas_call(
        _kl_body,
        out_shape=jax.ShapeDtypeStruct((1, 1), jnp.float32),
    )(matched, seg_pad)
    return out[0, 0]


def kernel(student_coords, teacher_coords, teacher_logits, seg_logits):
    q8 = jnp.pad(student_coords, ((0, 0), (0, 5)))
    t8 = jnp.pad(teacher_coords, ((0, 0), (0, 5))).T
    idx = _knn_idx(q8, t8)

    table = jnp.pad(teacher_logits, ((0, 0), (0, CP - C)))
    matched = _make_sc_gather()(table, idx)

    seg_pad = jnp.pad(seg_logits, ((0, 0), (0, 24 - C)))
    return _kl_loss(matched, seg_pad)
